# baseline (speedup 1.0000x reference)
import functools

import jax
import jax.numpy as jnp
from jax import lax
from jax.experimental import pallas as pl
from jax.experimental.pallas import tpu as pltpu
from jax.experimental.pallas import tpu_sc as plsc

_VOCAB = 1000000
_COL = 64
_PAD = 128
_BATCH = 4096
_SEQ = 200

_N = _BATCH * _SEQ
_NW = 32
_ROWS_W = _BATCH // _NW
_PER_W = _ROWS_W * _SEQ
_S1 = 128
_S2 = _SEQ - _S1
_NBUF = 4
_NJ = _ROWS_W * 2
_ITERS = _NJ // _NBUF


def _gather_body(table_hbm, x1_hbm, x2_hbm, out_hbm, idx1_v, idx2_v, *scratch):
    bufs = scratch[:_NBUF]
    gsems = scratch[_NBUF:2 * _NBUF]
    wsems = scratch[2 * _NBUF:]

    wid = lax.axis_index("s") * 2 + lax.axis_index("c")
    rbase = wid * _ROWS_W
    base = wid * _PER_W
    pltpu.sync_copy(x1_hbm.at[pl.ds(rbase, _ROWS_W)], idx1_v)
    pltpu.sync_copy(x2_hbm.at[pl.ds(rbase, _ROWS_W)], idx2_v)

    def _src(b, r):
        if b % 2 == 0:
            return idx1_v.at[r, pl.ds(0, _S1)], _S1, 0
        return idx2_v.at[r, pl.ds(0, _S2)], _S2, _S1

    def gstart(j, b):
        r = j // 2
        iv, size, off = _src(b, r)
        pltpu.async_copy(table_hbm.at[iv], bufs[b], gsems[b])

    def gwait(j, b):
        r = j // 2
        iv, size, off = _src(b, r)
        pltpu.make_async_copy(table_hbm.at[iv], bufs[b], gsems[b]).wait()

    def wstart(j, b):
        r = j // 2
        iv, size, off = _src(b, r)
        pltpu.async_copy(
            bufs[b],
            out_hbm.at[pl.ds(base + r * _SEQ + off, size)],
            wsems[b],
        )

    def wwait(j, b):
        r = j // 2
        iv, size, off = _src(b, r)
        pltpu.make_async_copy(
            bufs[b],
            out_hbm.at[pl.ds(base + r * _SEQ + off, size)],
            wsems[b],
        ).wait()

    for b in range(_NBUF):
        gstart(b, b)

    def body(it, _):
        for b in range(_NBUF):
            j = it * _NBUF + b
            gwait(j, b)
            wstart(j, b)

            @pl.when(it + 1 < _ITERS)
            def _():
                wwait(j, b)
                gstart(j + _NBUF, b)

        return 0

    lax.fori_loop(0, _ITERS, body, 0)

    for b in range(_NBUF):
        wwait((_ITERS - 1) * _NBUF + b, b)


def kernel(x, embed_weight):
    xi = x.astype(jnp.int32)
    x1 = xi[:, :_S1]
    x2 = jnp.pad(xi[:, _S1:], ((0, 0), (0, _PAD - _S2)))
    mesh = plsc.VectorSubcoreMesh(core_axis_name="c", subcore_axis_name="s")

    gather = functools.partial(
        pl.kernel,
        mesh=mesh,
        out_type=jax.ShapeDtypeStruct((_N, _PAD), jnp.float32),
        scratch_types=(
            [
                pltpu.VMEM((_ROWS_W, _S1), jnp.int32),
                pltpu.VMEM((_ROWS_W, _PAD), jnp.int32),
            ]
            + [
                pltpu.VMEM((_S1 if b % 2 == 0 else _S2, _PAD), jnp.float32)
                for b in range(_NBUF)
            ]
            + [pltpu.SemaphoreType.DMA for _ in range(2 * _NBUF)]
        ),
        compiler_params=pltpu.CompilerParams(use_tc_tiling_on_sc=False),
    )(_gather_body)

    tbl = jnp.pad(embed_weight, ((0, 0), (0, _PAD - _COL)))
    out = gather(tbl, x1, x2)
    return out[:, :_COL].reshape(_BATCH, _SEQ, _COL)


# speedup vs baseline: 1.0924x; 1.0924x over previous
import functools

import jax
import jax.numpy as jnp
from jax import lax
from jax.experimental import pallas as pl
from jax.experimental.pallas import tpu as pltpu
from jax.experimental.pallas import tpu_sc as plsc

_VOCAB = 1000000
_COL = 64
_PAD = 128
_BATCH = 4096
_SEQ = 200

_N = _BATCH * _SEQ
_NW = 32
_ROWS_W = _BATCH // _NW
_PER_W = _ROWS_W * _SEQ
_S1 = 128
_S2 = _SEQ - _S1
_NBUF = 4
_NJ = _ROWS_W * 2
_ITERS = _NJ // _NBUF


def _gather_body(table_hbm, x_hbm, out_hbm, idx_v, *scratch):
    bufs = scratch[:_NBUF]
    gsems = scratch[_NBUF:2 * _NBUF]
    wsems = scratch[2 * _NBUF:]

    wid = lax.axis_index("s") * 2 + lax.axis_index("c")
    rbase = wid * _ROWS_W
    base = wid * _PER_W
    pltpu.sync_copy(x_hbm.at[pl.ds(rbase, _ROWS_W)], idx_v)

    def _sz(b):
        return (_S1, 0) if b % 2 == 0 else (_S2, _S1)

    def gstart(j, b):
        size, off = _sz(b)
        r = j // 2
        pltpu.async_copy(
            table_hbm.at[idx_v.at[r, pl.ds(off, size)]], bufs[b], gsems[b]
        )

    def gwait(j, b):
        size, off = _sz(b)
        r = j // 2
        pltpu.make_async_copy(
            table_hbm.at[idx_v.at[r, pl.ds(off, size)]], bufs[b], gsems[b]
        ).wait()

    def wstart(j, b):
        size, off = _sz(b)
        r = j // 2
        pltpu.async_copy(
            bufs[b],
            out_hbm.at[pl.ds(base + r * _SEQ + off, size), pl.ds(0, _COL)],
            wsems[b],
        )

    def wwait(j, b):
        size, off = _sz(b)
        r = j // 2
        pltpu.make_async_copy(
            bufs[b],
            out_hbm.at[pl.ds(base + r * _SEQ + off, size), pl.ds(0, _COL)],
            wsems[b],
        ).wait()

    for b in range(_NBUF):
        gstart(b, b)

    def body(it, _):
        for b in range(_NBUF):
            j = it * _NBUF + b
            gwait(j, b)
            wstart(j, b)

            @pl.when(it + 1 < _ITERS)
            def _():
                wwait(j, b)
                gstart(j + _NBUF, b)

        return 0

    lax.fori_loop(0, _ITERS, body, 0)

    for b in range(_NBUF):
        wwait((_ITERS - 1) * _NBUF + b, b)


def kernel(x, embed_weight):
    xi = x.astype(jnp.int32)
    mesh = plsc.VectorSubcoreMesh(core_axis_name="c", subcore_axis_name="s")

    gather = functools.partial(
        pl.kernel,
        mesh=mesh,
        out_type=jax.ShapeDtypeStruct((_N, _PAD), jnp.float32),
        scratch_types=(
            [pltpu.VMEM((_ROWS_W, _SEQ), jnp.int32)]
            + [
                pltpu.VMEM((_S1 if b % 2 == 0 else _S2, _COL), jnp.float32)
                for b in range(_NBUF)
            ]
            + [pltpu.SemaphoreType.DMA for _ in range(2 * _NBUF)]
        ),
        compiler_params=pltpu.CompilerParams(use_tc_tiling_on_sc=False),
    )(_gather_body)

    out = gather(embed_weight, xi)
    return out[:, :_COL].reshape(_BATCH, _SEQ, _COL)
